# Initial kernel scaffold; baseline (speedup 1.0000x reference)
import jax
import jax.numpy as jnp
from jax.experimental import pallas as pl


def kernel(f_ir, f_vis, params):
    raise NotImplementedError("write your pallas kernel here")


# speedup vs baseline: 4.1591x; 4.1591x over previous
import jax
import jax.numpy as jnp
import numpy as np
from jax import lax
from jax.experimental import pallas as pl

_B, _C, _H, _W = 2, 96, 512, 512
_N = _H * _W
_K = 4096
_C2 = 48
_BH = 32
_NB = _H // _BH
_NH = 4
_HD = _C // _NH
_QB = 1024
_BNINV = float(1.0 / np.sqrt(1.0 + 1e-5))


def _a1_kernel(fir_ref, fvis_ref, w1_ref, g1_ref, b1_ref,
               fsum_ref, y1_ref, yfirst_ref, ylast_ref):
    fir = fir_ref[0]
    fvis = fvis_ref[0]
    fsum_ref[0] = fir + fvis
    x = jnp.concatenate([fir, fvis], axis=0).reshape(2 * _C, _BH * _W)
    y = lax.dot_general(w1_ref[...], x, (((1,), (0,)), ((), ())),
                        preferred_element_type=jnp.float32)
    y = y * (g1_ref[...] * _BNINV) + b1_ref[...]
    y = jnp.maximum(y, 0.0).reshape(_C2, _BH, _W)
    y1_ref[0] = y
    yfirst_ref[0, 0] = y[:, 0]
    ylast_ref[0, 0] = y[:, _BH - 1]


def _a2_kernel(y1_ref, prev_ref, nxt_ref, w2_ref, g2_ref, b2_ref,
               w3_ref, b3_ref, pred_ref, sal_ref):
    i = pl.program_id(1)
    y1 = y1_ref[0]
    prev = prev_ref[0, 0][:, None, :]
    nxt = nxt_ref[0, 0][:, None, :]
    top = jnp.where(i == 0, jnp.zeros_like(prev), prev)
    bot = jnp.where(i == _NB - 1, jnp.zeros_like(nxt), nxt)
    yh = jnp.concatenate([top, y1, bot], axis=1)
    zcol = jnp.zeros((_C2, _BH + 2, 1), jnp.float32)
    yhp = jnp.concatenate([zcol, yh, zcol], axis=2)
    acc = jnp.zeros((_C2, _BH, _W), jnp.float32)
    for di in range(3):
        for dj in range(3):
            wtap = w2_ref[:, 3 * di + dj].reshape(_C2, 1, 1)
            acc = acc + yhp[:, di:di + _BH, dj:dj + _W] * wtap
    y2 = acc.reshape(_C2, _BH * _W) * (g2_ref[...] * _BNINV) + b2_ref[...]
    y2 = jnp.maximum(y2, 0.0)
    z = lax.dot_general(w3_ref[...], y2, (((1,), (0,)), ((), ())),
                        preferred_element_type=jnp.float32)
    z = z + b3_ref[0, 0]
    pred = jax.nn.sigmoid(z).reshape(1, _BH, _W)
    pred_ref[...] = pred
    sal_ref[...] = jnp.abs(pred - 0.5)


def _ln(x, g, b):
    m = jnp.mean(x, axis=-1, keepdims=True)
    v = jnp.mean((x - m) ** 2, axis=-1, keepdims=True)
    return (x - m) / jnp.sqrt(v + 1e-5) * g + b


def _mixer_kernel(x_ref, ln1g, ln1b, inw, inb, outw, outb,
                  ln2g, ln2b, f1w, f1b, f2w, f2b, o_ref):
    x = x_ref[0, 0]
    xn = _ln(x, ln1g[0], ln1b[0])
    qkv = jnp.dot(xn, inw[0].T, preferred_element_type=jnp.float32) + inb[0]
    scale = float(1.0 / np.sqrt(_HD))
    heads_out = []
    for h in range(_NH):
        q = qkv[:, h * _HD:(h + 1) * _HD]
        k = qkv[:, _C + h * _HD:_C + (h + 1) * _HD]
        v = qkv[:, 2 * _C + h * _HD:2 * _C + (h + 1) * _HD]
        rows = []
        for qb in range(_K // _QB):
            qs = q[qb * _QB:(qb + 1) * _QB]
            s = jnp.dot(qs, k.T, preferred_element_type=jnp.float32) * scale
            s = s - jnp.max(s, axis=-1, keepdims=True)
            e = jnp.exp(s)
            p = e / jnp.sum(e, axis=-1, keepdims=True)
            rows.append(jnp.dot(p, v, preferred_element_type=jnp.float32))
        heads_out.append(jnp.concatenate(rows, axis=0))
    o = jnp.concatenate(heads_out, axis=1)
    x = x + jnp.dot(o, outw[0].T, preferred_element_type=jnp.float32) + outb[0]
    xn2 = _ln(x, ln2g[0], ln2b[0])
    hdn = jnp.dot(xn2, f1w[0].T, preferred_element_type=jnp.float32) + f1b[0]
    hdn = 0.5 * hdn * (1.0 + lax.erf(hdn * float(1.0 / np.sqrt(2.0))))
    x = x + jnp.dot(hdn, f2w[0].T, preferred_element_type=jnp.float32) + f2b[0]
    o_ref[0, 0] = x


def kernel(f_ir, f_vis, params):
    p = params
    f_ir = f_ir.astype(jnp.float32)
    f_vis = f_vis.astype(jnp.float32)
    w1 = p['a_c1'].reshape(_C2, 2 * _C)
    g1 = p['a_bn1_g'].reshape(_C2, 1)
    b1 = p['a_bn1_b'].reshape(_C2, 1)
    w2 = p['a_c2'].reshape(_C2, 9)
    g2 = p['a_bn2_g'].reshape(_C2, 1)
    b2 = p['a_bn2_b'].reshape(_C2, 1)
    w3 = p['a_c3'].reshape(1, _C2)
    b3 = p['a_c3_b'].reshape(1, 1)

    grid = (_B, _NB)
    fsum, y1, yfirst, ylast = pl.pallas_call(
        _a1_kernel,
        grid=grid,
        in_specs=[
            pl.BlockSpec((1, _C, _BH, _W), lambda b, i: (b, 0, i, 0)),
            pl.BlockSpec((1, _C, _BH, _W), lambda b, i: (b, 0, i, 0)),
            pl.BlockSpec((_C2, 2 * _C), lambda b, i: (0, 0)),
            pl.BlockSpec((_C2, 1), lambda b, i: (0, 0)),
            pl.BlockSpec((_C2, 1), lambda b, i: (0, 0)),
        ],
        out_specs=[
            pl.BlockSpec((1, _C, _BH, _W), lambda b, i: (b, 0, i, 0)),
            pl.BlockSpec((1, _C2, _BH, _W), lambda b, i: (b, 0, i, 0)),
            pl.BlockSpec((1, 1, _C2, _W), lambda b, i: (b, i, 0, 0)),
            pl.BlockSpec((1, 1, _C2, _W), lambda b, i: (b, i, 0, 0)),
        ],
        out_shape=[
            jax.ShapeDtypeStruct((_B, _C, _H, _W), jnp.float32),
            jax.ShapeDtypeStruct((_B, _C2, _H, _W), jnp.float32),
            jax.ShapeDtypeStruct((_B, _NB, _C2, _W), jnp.float32),
            jax.ShapeDtypeStruct((_B, _NB, _C2, _W), jnp.float32),
        ],
    )(f_ir, f_vis, w1, g1, b1)

    pred, sal = pl.pallas_call(
        _a2_kernel,
        grid=grid,
        in_specs=[
            pl.BlockSpec((1, _C2, _BH, _W), lambda b, i: (b, 0, i, 0)),
            pl.BlockSpec((1, 1, _C2, _W),
                         lambda b, i: (b, jnp.maximum(i - 1, 0), 0, 0)),
            pl.BlockSpec((1, 1, _C2, _W),
                         lambda b, i: (b, jnp.minimum(i + 1, _NB - 1), 0, 0)),
            pl.BlockSpec((_C2, 9), lambda b, i: (0, 0)),
            pl.BlockSpec((_C2, 1), lambda b, i: (0, 0)),
            pl.BlockSpec((_C2, 1), lambda b, i: (0, 0)),
            pl.BlockSpec((1, _C2), lambda b, i: (0, 0)),
            pl.BlockSpec((1, 1), lambda b, i: (0, 0)),
        ],
        out_specs=[
            pl.BlockSpec((1, _BH, _W), lambda b, i: (b, i, 0)),
            pl.BlockSpec((1, _BH, _W), lambda b, i: (b, i, 0)),
        ],
        out_shape=[
            jax.ShapeDtypeStruct((_B, _H, _W), jnp.float32),
            jax.ShapeDtypeStruct((_B, _H, _W), jnp.float32),
        ],
    )(y1, ylast, yfirst, w2, g2, b2, w3, b3)

    sal_flat = sal.reshape(_B, _N)
    _, idx = lax.top_k(sal_flat, _K)

    fir_cm = f_ir.reshape(_B, _C, _N)
    fvis_cm = f_vis.reshape(_B, _C, _N)
    gi = jnp.broadcast_to(idx[:, None, :], (_B, _C, _K))
    sel_ir = jnp.take_along_axis(fir_cm, gi, axis=2).transpose(0, 2, 1)
    sel_vis = jnp.take_along_axis(fvis_cm, gi, axis=2).transpose(0, 2, 1)
    X = jnp.stack([sel_ir, sel_vis])

    def mstack(name, shape):
        return jnp.stack([p['mir_' + name], p['mvis_' + name]]).reshape(shape)

    ln1g = mstack('ln1_g', (2, 1, _C))
    ln1b = mstack('ln1_b', (2, 1, _C))
    inw = mstack('in_w', (2, 3 * _C, _C))
    inb = mstack('in_b', (2, 1, 3 * _C))
    outw = mstack('out_w', (2, _C, _C))
    outb = mstack('out_b', (2, 1, _C))
    ln2g = mstack('ln2_g', (2, 1, _C))
    ln2b = mstack('ln2_b', (2, 1, _C))
    f1w = mstack('f1_w', (2, 4 * _C, _C))
    f1b = mstack('f1_b', (2, 1, 4 * _C))
    f2w = mstack('f2_w', (2, _C, 4 * _C))
    f2b = mstack('f2_b', (2, 1, _C))

    def wspec(shape):
        return pl.BlockSpec((1,) + shape[1:], lambda s, b: (s, 0, 0))

    O = pl.pallas_call(
        _mixer_kernel,
        grid=(2, _B),
        in_specs=[
            pl.BlockSpec((1, 1, _K, _C), lambda s, b: (s, b, 0, 0)),
            wspec(ln1g.shape), wspec(ln1b.shape),
            wspec(inw.shape), wspec(inb.shape),
            wspec(outw.shape), wspec(outb.shape),
            wspec(ln2g.shape), wspec(ln2b.shape),
            wspec(f1w.shape), wspec(f1b.shape),
            wspec(f2w.shape), wspec(f2b.shape),
        ],
        out_specs=pl.BlockSpec((1, 1, _K, _C), lambda s, b: (s, b, 0, 0)),
        out_shape=jax.ShapeDtypeStruct((2, _B, _K, _C), jnp.float32),
    )(X, ln1g, ln1b, inw, inb, outw, outb, ln2g, ln2b, f1w, f1b, f2w, f2b)

    sel_w = jnp.take_along_axis(pred.reshape(_B, _N), idx, axis=1)[:, :, None]
    fused = O[0] * sel_w + O[1] * (1.0 - sel_w)
    fused_cm = fused.transpose(0, 2, 1)
    bb = jnp.arange(_B)[:, None, None]
    cc = jnp.arange(_C)[None, :, None]
    f_final = fsum.reshape(_B, _C, _N).at[bb, cc, idx[:, None, :]].set(fused_cm)
    return (f_final.reshape(_B, _C, _H, _W),
            jnp.asarray(0.0, jnp.float32))


# speedup vs baseline: 6.0077x; 1.4445x over previous
import jax
import jax.numpy as jnp
import numpy as np
from jax import lax
from jax.experimental import pallas as pl

_B, _C, _H, _W = 2, 96, 512, 512
_N = _H * _W
_K = 4096
_C2 = 48
_BH = 32
_NB = _H // _BH
_NH = 4
_HD = _C // _NH
_QB = 1024
_BNINV = float(1.0 / np.sqrt(1.0 + 1e-5))


def _a1_kernel(fir_ref, fvis_ref, w1_ref, g1_ref, b1_ref,
               fsum_ref, y1_ref, yfirst_ref, ylast_ref):
    fir = fir_ref[0]
    fvis = fvis_ref[0]
    fsum_ref[0] = fir + fvis
    x = jnp.concatenate([fir, fvis], axis=0).reshape(2 * _C, _BH * _W)
    y = lax.dot_general(w1_ref[...], x, (((1,), (0,)), ((), ())),
                        preferred_element_type=jnp.float32)
    y = y * (g1_ref[...] * _BNINV) + b1_ref[...]
    y = jnp.maximum(y, 0.0).reshape(_C2, _BH, _W)
    y1_ref[0] = y
    yfirst_ref[0, 0] = y[:, 0]
    ylast_ref[0, 0] = y[:, _BH - 1]


def _a2_kernel(y1_ref, prev_ref, nxt_ref, w2_ref, g2_ref, b2_ref,
               w3_ref, b3_ref, pred_ref, sal_ref):
    i = pl.program_id(1)
    y1 = y1_ref[0]
    prev = prev_ref[0, 0][:, None, :]
    nxt = nxt_ref[0, 0][:, None, :]
    top = jnp.where(i == 0, jnp.zeros_like(prev), prev)
    bot = jnp.where(i == _NB - 1, jnp.zeros_like(nxt), nxt)
    yh = jnp.concatenate([top, y1, bot], axis=1)
    zcol = jnp.zeros((_C2, _BH + 2, 1), jnp.float32)
    yhp = jnp.concatenate([zcol, yh, zcol], axis=2)
    acc = jnp.zeros((_C2, _BH, _W), jnp.float32)
    for di in range(3):
        for dj in range(3):
            wtap = w2_ref[:, 3 * di + dj].reshape(_C2, 1, 1)
            acc = acc + yhp[:, di:di + _BH, dj:dj + _W] * wtap
    y2 = acc.reshape(_C2, _BH * _W) * (g2_ref[...] * _BNINV) + b2_ref[...]
    y2 = jnp.maximum(y2, 0.0)
    z = lax.dot_general(w3_ref[...], y2, (((1,), (0,)), ((), ())),
                        preferred_element_type=jnp.float32)
    z = z + b3_ref[0, 0]
    pred = jax.nn.sigmoid(z).reshape(1, _BH, _W)
    pred_ref[...] = pred
    sal_ref[...] = jnp.abs(pred - 0.5)


def _ln(x, g, b):
    m = jnp.mean(x, axis=-1, keepdims=True)
    v = jnp.mean((x - m) ** 2, axis=-1, keepdims=True)
    return (x - m) / jnp.sqrt(v + 1e-5) * g + b


def _mixer_kernel(x_ref, ln1g, ln1b, inw, inb, outw, outb,
                  ln2g, ln2b, f1w, f1b, f2w, f2b, o_ref):
    x = x_ref[0, 0]
    xn = _ln(x, ln1g[0], ln1b[0])
    qkv = jnp.dot(xn, inw[0].T, preferred_element_type=jnp.float32) + inb[0]
    scale = float(1.0 / np.sqrt(_HD))
    heads_out = []
    for h in range(_NH):
        q = qkv[:, h * _HD:(h + 1) * _HD]
        k = qkv[:, _C + h * _HD:_C + (h + 1) * _HD]
        v = qkv[:, 2 * _C + h * _HD:2 * _C + (h + 1) * _HD]
        rows = []
        for qb in range(_K // _QB):
            qs = q[qb * _QB:(qb + 1) * _QB]
            s = jnp.dot(qs, k.T, preferred_element_type=jnp.float32) * scale
            s = s - jnp.max(s, axis=-1, keepdims=True)
            e = jnp.exp(s)
            p = e / jnp.sum(e, axis=-1, keepdims=True)
            rows.append(jnp.dot(p, v, preferred_element_type=jnp.float32))
        heads_out.append(jnp.concatenate(rows, axis=0))
    o = jnp.concatenate(heads_out, axis=1)
    x = x + jnp.dot(o, outw[0].T, preferred_element_type=jnp.float32) + outb[0]
    xn2 = _ln(x, ln2g[0], ln2b[0])
    hdn = jnp.dot(xn2, f1w[0].T, preferred_element_type=jnp.float32) + f1b[0]
    hdn = 0.5 * hdn * (1.0 + lax.erf(hdn * float(1.0 / np.sqrt(2.0))))
    x = x + jnp.dot(hdn, f2w[0].T, preferred_element_type=jnp.float32) + f2b[0]
    o_ref[0, 0] = x


def kernel(f_ir, f_vis, params):
    p = params
    f_ir = f_ir.astype(jnp.float32)
    f_vis = f_vis.astype(jnp.float32)
    w1 = p['a_c1'].reshape(_C2, 2 * _C)
    g1 = p['a_bn1_g'].reshape(_C2, 1)
    b1 = p['a_bn1_b'].reshape(_C2, 1)
    w2 = p['a_c2'].reshape(_C2, 9)
    g2 = p['a_bn2_g'].reshape(_C2, 1)
    b2 = p['a_bn2_b'].reshape(_C2, 1)
    w3 = p['a_c3'].reshape(1, _C2)
    b3 = p['a_c3_b'].reshape(1, 1)

    grid = (_B, _NB)
    fsum, y1, yfirst, ylast = pl.pallas_call(
        _a1_kernel,
        grid=grid,
        in_specs=[
            pl.BlockSpec((1, _C, _BH, _W), lambda b, i: (b, 0, i, 0)),
            pl.BlockSpec((1, _C, _BH, _W), lambda b, i: (b, 0, i, 0)),
            pl.BlockSpec((_C2, 2 * _C), lambda b, i: (0, 0)),
            pl.BlockSpec((_C2, 1), lambda b, i: (0, 0)),
            pl.BlockSpec((_C2, 1), lambda b, i: (0, 0)),
        ],
        out_specs=[
            pl.BlockSpec((1, _C, _BH, _W), lambda b, i: (b, 0, i, 0)),
            pl.BlockSpec((1, _C2, _BH, _W), lambda b, i: (b, 0, i, 0)),
            pl.BlockSpec((1, 1, _C2, _W), lambda b, i: (b, i, 0, 0)),
            pl.BlockSpec((1, 1, _C2, _W), lambda b, i: (b, i, 0, 0)),
        ],
        out_shape=[
            jax.ShapeDtypeStruct((_B, _C, _H, _W), jnp.float32),
            jax.ShapeDtypeStruct((_B, _C2, _H, _W), jnp.float32),
            jax.ShapeDtypeStruct((_B, _NB, _C2, _W), jnp.float32),
            jax.ShapeDtypeStruct((_B, _NB, _C2, _W), jnp.float32),
        ],
    )(f_ir, f_vis, w1, g1, b1)

    pred, sal = pl.pallas_call(
        _a2_kernel,
        grid=grid,
        in_specs=[
            pl.BlockSpec((1, _C2, _BH, _W), lambda b, i: (b, 0, i, 0)),
            pl.BlockSpec((1, 1, _C2, _W),
                         lambda b, i: (b, jnp.maximum(i - 1, 0), 0, 0)),
            pl.BlockSpec((1, 1, _C2, _W),
                         lambda b, i: (b, jnp.minimum(i + 1, _NB - 1), 0, 0)),
            pl.BlockSpec((_C2, 9), lambda b, i: (0, 0)),
            pl.BlockSpec((_C2, 1), lambda b, i: (0, 0)),
            pl.BlockSpec((_C2, 1), lambda b, i: (0, 0)),
            pl.BlockSpec((1, _C2), lambda b, i: (0, 0)),
            pl.BlockSpec((1, 1), lambda b, i: (0, 0)),
        ],
        out_specs=[
            pl.BlockSpec((1, _BH, _W), lambda b, i: (b, i, 0)),
            pl.BlockSpec((1, _BH, _W), lambda b, i: (b, i, 0)),
        ],
        out_shape=[
            jax.ShapeDtypeStruct((_B, _H, _W), jnp.float32),
            jax.ShapeDtypeStruct((_B, _H, _W), jnp.float32),
        ],
    )(y1, ylast, yfirst, w2, g2, b2, w3, b3)

    sal_flat = sal.reshape(_B, _N)
    _, idx = lax.top_k(sal_flat, _K)

    fir_cm = f_ir.reshape(_B, _C, _N)
    fvis_cm = f_vis.reshape(_B, _C, _N)
    bsel = jnp.arange(_B)[:, None]
    sel_ir = fir_cm[bsel, :, idx]
    sel_vis = fvis_cm[bsel, :, idx]
    X = jnp.stack([sel_ir, sel_vis])

    def mstack(name, shape):
        return jnp.stack([p['mir_' + name], p['mvis_' + name]]).reshape(shape)

    ln1g = mstack('ln1_g', (2, 1, _C))
    ln1b = mstack('ln1_b', (2, 1, _C))
    inw = mstack('in_w', (2, 3 * _C, _C))
    inb = mstack('in_b', (2, 1, 3 * _C))
    outw = mstack('out_w', (2, _C, _C))
    outb = mstack('out_b', (2, 1, _C))
    ln2g = mstack('ln2_g', (2, 1, _C))
    ln2b = mstack('ln2_b', (2, 1, _C))
    f1w = mstack('f1_w', (2, 4 * _C, _C))
    f1b = mstack('f1_b', (2, 1, 4 * _C))
    f2w = mstack('f2_w', (2, _C, 4 * _C))
    f2b = mstack('f2_b', (2, 1, _C))

    def wspec(shape):
        return pl.BlockSpec((1,) + shape[1:], lambda s, b: (s, 0, 0))

    O = pl.pallas_call(
        _mixer_kernel,
        grid=(2, _B),
        in_specs=[
            pl.BlockSpec((1, 1, _K, _C), lambda s, b: (s, b, 0, 0)),
            wspec(ln1g.shape), wspec(ln1b.shape),
            wspec(inw.shape), wspec(inb.shape),
            wspec(outw.shape), wspec(outb.shape),
            wspec(ln2g.shape), wspec(ln2b.shape),
            wspec(f1w.shape), wspec(f1b.shape),
            wspec(f2w.shape), wspec(f2b.shape),
        ],
        out_specs=pl.BlockSpec((1, 1, _K, _C), lambda s, b: (s, b, 0, 0)),
        out_shape=jax.ShapeDtypeStruct((2, _B, _K, _C), jnp.float32),
    )(X, ln1g, ln1b, inw, inb, outw, outb, ln2g, ln2b, f1w, f1b, f2w, f2b)

    sel_w = jnp.take_along_axis(pred.reshape(_B, _N), idx, axis=1)[:, :, None]
    fused = O[0] * sel_w + O[1] * (1.0 - sel_w)
    f_final = fsum.reshape(_B, _C, _N).at[bsel, :, idx].set(fused)
    return (f_final.reshape(_B, _C, _H, _W),
            jnp.asarray(0.0, jnp.float32))
